# baseline (speedup 1.0000x reference)
import functools

import jax
import jax.numpy as jnp
from jax import lax
from jax.experimental import pallas as pl
from jax.experimental.pallas import tpu as pltpu
from jax.experimental.pallas import tpu_sc as plsc

DIM = 64
EPS = 1e-12
B, S = 4096, 200
NC, NS = 2, 16
NW = NC * NS
CHUNK_SEQ = 2
CT = CHUNK_SEQ * S
NCHUNK = B // CHUNK_SEQ
CPW = NCHUNK // NW
NP = CPW // 2
NIDX = 4
IDXW = CT // NIDX
LANES = 16
NV = DIM // LANES


def _rsqrt(v):
    vi = lax.bitcast_convert_type(v, jnp.int32)
    yi = jnp.int32(0x5F3759DF) - lax.shift_right_arithmetic(vi, 1)
    y = lax.bitcast_convert_type(yi, jnp.float32)
    h = v * 0.5
    for _ in range(3):
        y = y * (1.5 - h * y * y)
    return y


def kernel(x, W, P, gamma, beta):
    x = x.astype(jnp.int32).reshape(NCHUNK, NIDX, IDXW)
    mesh = plsc.VectorSubcoreMesh(core_axis_name="c", subcore_axis_name="s")

    @functools.partial(
        pl.kernel,
        out_type=jax.ShapeDtypeStruct((NCHUNK, CT, DIM), jnp.float32),
        mesh=mesh,
        scratch_types=[
            pltpu.VMEM((NIDX, IDXW), jnp.int32),
            pltpu.VMEM((NIDX, IDXW), jnp.int32),
            pltpu.VMEM((CT, DIM), jnp.float32),
            pltpu.VMEM((CT, DIM), jnp.float32),
            pltpu.VMEM((S, DIM), jnp.float32),
            pltpu.VMEM((DIM,), jnp.float32),
            pltpu.VMEM((DIM,), jnp.float32),
            pltpu.VMEM((2 * CT,), jnp.float32),
            pltpu.SemaphoreType.DMA,
            pltpu.SemaphoreType.DMA,
            pltpu.SemaphoreType.DMA,
            pltpu.SemaphoreType.DMA,
        ],
        compiler_params=pltpu.CompilerParams(needs_layout_passes=False,
                                             use_tc_tiling_on_sc=False),
    )
    def sc_fn(x_hbm, w_hbm, p_hbm, g_hbm, b_hbm, out_hbm,
              idx0, idx1, rows0, rows1, p_v, g_v, b_v, acc_v,
              gsem0, gsem1, osem0, osem1):
        wid = lax.axis_index("s") * NC + lax.axis_index("c")
        base = wid * CPW
        pltpu.sync_copy(p_hbm, p_v)
        pltpu.sync_copy(g_hbm, g_v)
        pltpu.sync_copy(b_hbm, b_v)
        g_regs = [g_v[pl.ds(c * LANES, LANES)] for c in range(NV)]
        b_regs = [b_v[pl.ds(c * LANES, LANES)] for c in range(NV)]
        zerov = jnp.zeros((LANES,), jnp.float32)

        def fire_gather(idx_b, rows_b, sem):
            for j in range(NIDX):
                pltpu.async_copy(w_hbm.at[idx_b.at[j]],
                                 rows_b.at[pl.ds(j * IDXW, IDXW)], sem)

        def drain_gather(idx_b, rows_b, sem):
            for j in range(NIDX):
                pltpu.make_async_copy(w_hbm.at[idx_b.at[j]],
                                      rows_b.at[pl.ds(j * IDXW, IDXW)],
                                      sem).wait()

        def compute(rows_b):
            for j in range(2 * CT // LANES):
                acc_v[pl.ds(j * LANES, LANES)] = zerov

            @plsc.parallel_loop(0, S, unroll=8)
            def tok_body(si):
                for r2 in range(CHUNK_SEQ):
                    t = r2 * S + si
                    e = [rows_b[t, pl.ds(c * LANES, LANES)]
                         + p_v[si, pl.ds(c * LANES, LANES)]
                         for c in range(NV)]
                    s4 = (e[0] + e[1]) + (e[2] + e[3])
                    q4 = (e[0] * e[0] + e[1] * e[1]) + (e[2] * e[2] + e[3] * e[3])
                    ti = jnp.full((LANES,), t, jnp.int32)
                    qi = ti + CT
                    plsc.addupdate_scatter(acc_v, [ti], s4)
                    plsc.addupdate_scatter(acc_v, [qi], q4)
                    ssum = plsc.load_gather(acc_v, [ti])
                    qsum = plsc.load_gather(acc_v, [qi])
                    mean = ssum * (1.0 / DIM)
                    var = qsum * (1.0 / DIM) - mean * mean
                    rstd = _rsqrt(var + EPS)
                    for c in range(NV):
                        rows_b[t, pl.ds(c * LANES, LANES)] = (
                            (e[c] - mean) * rstd * g_regs[c] + b_regs[c])

        pltpu.sync_copy(x_hbm.at[base], idx0)
        fire_gather(idx0, rows0, gsem0)

        def pair_body(p, carry):
            cA = base + 2 * p
            cB = cA + 1
            drain_gather(idx0, rows0, gsem0)

            @pl.when(p > 0)
            def _():
                pltpu.make_async_copy(rows1, out_hbm.at[cB - 2], osem1).wait()

            pltpu.sync_copy(x_hbm.at[cB], idx1)
            fire_gather(idx1, rows1, gsem1)
            compute(rows0)
            pltpu.async_copy(rows0, out_hbm.at[cA], osem0)
            drain_gather(idx1, rows1, gsem1)

            @pl.when(p < NP - 1)
            def _():
                pltpu.make_async_copy(rows0, out_hbm.at[cA], osem0).wait()
                pltpu.sync_copy(x_hbm.at[cA + 2], idx0)
                fire_gather(idx0, rows0, gsem0)

            compute(rows1)
            pltpu.async_copy(rows1, out_hbm.at[cB], osem1)
            return carry

        lax.fori_loop(0, NP, pair_body, 0)
        pltpu.make_async_copy(rows0, out_hbm.at[base + CPW - 2], osem0).wait()
        pltpu.make_async_copy(rows1, out_hbm.at[base + CPW - 1], osem1).wait()

    out = sc_fn(x, W, P, gamma, beta)
    return out.reshape(B, S, DIM)


# speedup vs baseline: 1.8067x; 1.8067x over previous
import functools

import jax
import jax.numpy as jnp
from jax import lax
from jax.experimental import pallas as pl
from jax.experimental.pallas import tpu as pltpu
from jax.experimental.pallas import tpu_sc as plsc

DIM = 64
EPS = 1e-12
B, S = 4096, 200
NC, NS = 2, 16
NW = NC * NS
CHUNK_SEQ = 2
CT = CHUNK_SEQ * S
NCHUNK = B // CHUNK_SEQ
CPW = NCHUNK // NW
NP = CPW // 2
NIDX = 4
IDXW = CT // NIDX
LANES = 16
NV = DIM // LANES


def _rsqrt(v):
    vi = lax.bitcast_convert_type(v, jnp.int32)
    yi = jnp.int32(0x5F3759DF) - lax.shift_right_arithmetic(vi, 1)
    y = lax.bitcast_convert_type(yi, jnp.float32)
    h = v * 0.5
    for _ in range(3):
        y = y * (1.5 - h * y * y)
    return y


def kernel(x, W, P, gamma, beta):
    x = x.astype(jnp.int32).reshape(NCHUNK, NIDX, IDXW)
    mesh = plsc.VectorSubcoreMesh(core_axis_name="c", subcore_axis_name="s")

    @functools.partial(
        pl.kernel,
        out_type=jax.ShapeDtypeStruct((NCHUNK, CT, DIM), jnp.float32),
        mesh=mesh,
        scratch_types=[
            pltpu.VMEM((NIDX, IDXW), jnp.int32),
            pltpu.VMEM((NIDX, IDXW), jnp.int32),
            pltpu.VMEM((CT, DIM), jnp.float32),
            pltpu.VMEM((CT, DIM), jnp.float32),
            pltpu.VMEM((S, DIM), jnp.float32),
            pltpu.VMEM((DIM,), jnp.float32),
            pltpu.VMEM((DIM,), jnp.float32),
            pltpu.VMEM((CT * 17,), jnp.float32),
            pltpu.VMEM((CT * 17,), jnp.float32),
            pltpu.VMEM((CT + LANES,), jnp.float32),
            pltpu.VMEM((CT + LANES,), jnp.float32),
            pltpu.SemaphoreType.DMA,
            pltpu.SemaphoreType.DMA,
            pltpu.SemaphoreType.DMA,
            pltpu.SemaphoreType.DMA,
        ],
        compiler_params=pltpu.CompilerParams(needs_layout_passes=False,
                                             use_tc_tiling_on_sc=False),
    )
    def sc_fn(x_hbm, w_hbm, p_hbm, g_hbm, b_hbm, out_hbm,
              idx0, idx1, rows0, rows1, p_v, g_v, b_v,
              sbuf, qbuf, macc, racc,
              gsem0, gsem1, osem0, osem1):
        wid = lax.axis_index("s") * NC + lax.axis_index("c")
        base = wid * CPW
        pltpu.sync_copy(p_hbm, p_v)
        pltpu.sync_copy(g_hbm, g_v)
        pltpu.sync_copy(b_hbm, b_v)
        g_regs = [g_v[pl.ds(c * LANES, LANES)] for c in range(NV)]
        b_regs = [b_v[pl.ds(c * LANES, LANES)] for c in range(NV)]

        def fire_gather(idx_b, rows_b, sem):
            for j in range(NIDX):
                pltpu.async_copy(w_hbm.at[idx_b.at[j]],
                                 rows_b.at[pl.ds(j * IDXW, IDXW)], sem)

        def drain_gather(idx_b, rows_b, sem):
            for j in range(NIDX):
                pltpu.make_async_copy(w_hbm.at[idx_b.at[j]],
                                      rows_b.at[pl.ds(j * IDXW, IDXW)],
                                      sem).wait()

        iota17 = lax.iota(jnp.int32, LANES) * 17

        def compute(rows_b):
            @plsc.parallel_loop(0, S, unroll=8)
            def pass1(si):
                for r2 in range(CHUNK_SEQ):
                    t = r2 * S + si
                    e = [rows_b[t, pl.ds(c * LANES, LANES)]
                         + p_v[si, pl.ds(c * LANES, LANES)]
                         for c in range(NV)]
                    for c in range(NV):
                        rows_b[t, pl.ds(c * LANES, LANES)] = e[c]
                    s4 = (e[0] + e[1]) + (e[2] + e[3])
                    q4 = (e[0] * e[0] + e[1] * e[1]) + (e[2] * e[2] + e[3] * e[3])
                    sbuf[pl.ds(t * 17, LANES)] = s4
                    qbuf[pl.ds(t * 17, LANES)] = q4

            @plsc.parallel_loop(0, CT // LANES, unroll=2)
            def stats(k):
                bvec = iota17 + k * (LANES * 17)
                stot = plsc.load_gather(sbuf, [bvec])
                qtot = plsc.load_gather(qbuf, [bvec])
                for j in range(1, LANES):
                    stot = stot + plsc.load_gather(sbuf, [bvec + j])
                    qtot = qtot + plsc.load_gather(qbuf, [bvec + j])
                mean = stot * (1.0 / DIM)
                var = qtot * (1.0 / DIM) - mean * mean
                rstd = _rsqrt(var + EPS)
                macc[pl.ds(k * LANES, LANES)] = mean
                racc[pl.ds(k * LANES, LANES)] = rstd

            @plsc.parallel_loop(0, S, unroll=8)
            def pass2(si):
                for r2 in range(CHUNK_SEQ):
                    t = r2 * S + si
                    mv = jnp.full((LANES,), macc[pl.ds(t, LANES)][0], jnp.float32)
                    rv = jnp.full((LANES,), racc[pl.ds(t, LANES)][0], jnp.float32)
                    for c in range(NV):
                        e = rows_b[t, pl.ds(c * LANES, LANES)]
                        rows_b[t, pl.ds(c * LANES, LANES)] = (
                            (e - mv) * rv * g_regs[c] + b_regs[c])

        pltpu.sync_copy(x_hbm.at[base], idx0)
        fire_gather(idx0, rows0, gsem0)

        def pair_body(p, carry):
            cA = base + 2 * p
            cB = cA + 1
            drain_gather(idx0, rows0, gsem0)

            @pl.when(p > 0)
            def _():
                pltpu.make_async_copy(rows1, out_hbm.at[cB - 2], osem1).wait()

            pltpu.sync_copy(x_hbm.at[cB], idx1)
            fire_gather(idx1, rows1, gsem1)
            compute(rows0)
            pltpu.async_copy(rows0, out_hbm.at[cA], osem0)
            drain_gather(idx1, rows1, gsem1)

            @pl.when(p < NP - 1)
            def _():
                pltpu.make_async_copy(rows0, out_hbm.at[cA], osem0).wait()
                pltpu.sync_copy(x_hbm.at[cA + 2], idx0)
                fire_gather(idx0, rows0, gsem0)

            compute(rows1)
            pltpu.async_copy(rows1, out_hbm.at[cB], osem1)
            return carry

        lax.fori_loop(0, NP, pair_body, 0)
        pltpu.make_async_copy(rows0, out_hbm.at[base + CPW - 2], osem0).wait()
        pltpu.make_async_copy(rows1, out_hbm.at[base + CPW - 1], osem1).wait()

    out = sc_fn(x, W, P, gamma, beta)
    return out.reshape(B, S, DIM)


# speedup vs baseline: 1.8931x; 1.0478x over previous
import functools

import jax
import jax.numpy as jnp
from jax import lax
from jax.experimental import pallas as pl
from jax.experimental.pallas import tpu as pltpu
from jax.experimental.pallas import tpu_sc as plsc

DIM = 64
EPS = 1e-12
B, S = 4096, 200
NC, NS = 2, 16
NW = NC * NS
CHUNK_SEQ = 2
CT = CHUNK_SEQ * S
NCHUNK = B // CHUNK_SEQ
CPW = NCHUNK // NW
NP = CPW // 2
NIDX = 4
IDXW = CT // NIDX
LANES = 16
NV = DIM // LANES


def _rsqrt(v):
    vi = lax.bitcast_convert_type(v, jnp.int32)
    yi = jnp.int32(0x5F3759DF) - lax.shift_right_arithmetic(vi, 1)
    y = lax.bitcast_convert_type(yi, jnp.float32)
    h = v * 0.5
    for _ in range(3):
        y = y * (1.5 - h * y * y)
    return y


def kernel(x, W, P, gamma, beta):
    x = x.astype(jnp.int32).reshape(NCHUNK, NIDX, IDXW)
    mesh = plsc.VectorSubcoreMesh(core_axis_name="c", subcore_axis_name="s")

    @functools.partial(
        pl.kernel,
        out_type=jax.ShapeDtypeStruct((NCHUNK, CT, DIM), jnp.float32),
        mesh=mesh,
        scratch_types=[
            pltpu.VMEM((NIDX, IDXW), jnp.int32),
            pltpu.VMEM((NIDX, IDXW), jnp.int32),
            pltpu.VMEM((CT, DIM), jnp.float32),
            pltpu.VMEM((CT, DIM), jnp.float32),
            pltpu.VMEM((S, DIM), jnp.float32),
            pltpu.VMEM((DIM,), jnp.float32),
            pltpu.VMEM((DIM,), jnp.float32),
            pltpu.VMEM((CT * 17,), jnp.float32),
            pltpu.VMEM((CT * 17,), jnp.float32),
            pltpu.SemaphoreType.DMA,
            pltpu.SemaphoreType.DMA,
            pltpu.SemaphoreType.DMA,
            pltpu.SemaphoreType.DMA,
        ],
        compiler_params=pltpu.CompilerParams(needs_layout_passes=False,
                                             use_tc_tiling_on_sc=False),
    )
    def sc_fn(x_hbm, w_hbm, p_hbm, g_hbm, b_hbm, out_hbm,
              idx0, idx1, rows0, rows1, p_v, g_v, b_v,
              sbuf, qbuf,
              gsem0, gsem1, osem0, osem1):
        wid = lax.axis_index("s") * NC + lax.axis_index("c")
        base = wid * CPW
        pltpu.sync_copy(p_hbm, p_v)
        pltpu.sync_copy(g_hbm, g_v)
        pltpu.sync_copy(b_hbm, b_v)
        g_regs = [g_v[pl.ds(c * LANES, LANES)] for c in range(NV)]
        b_regs = [b_v[pl.ds(c * LANES, LANES)] for c in range(NV)]

        def fire_gather(idx_b, rows_b, sem):
            for j in range(NIDX):
                pltpu.async_copy(w_hbm.at[idx_b.at[j]],
                                 rows_b.at[pl.ds(j * IDXW, IDXW)], sem)

        def drain_gather(idx_b, rows_b, sem):
            for j in range(NIDX):
                pltpu.make_async_copy(w_hbm.at[idx_b.at[j]],
                                      rows_b.at[pl.ds(j * IDXW, IDXW)],
                                      sem).wait()

        iota17 = lax.iota(jnp.int32, LANES) * 17

        def compute(rows_b):
            @plsc.parallel_loop(0, S, unroll=8)
            def pass1(si):
                p = [p_v[si, pl.ds(c * LANES, LANES)] for c in range(NV)]
                for r2 in range(CHUNK_SEQ):
                    t = r2 * S + si
                    e = [rows_b[t, pl.ds(c * LANES, LANES)] + p[c]
                         for c in range(NV)]
                    for c in range(NV):
                        rows_b[t, pl.ds(c * LANES, LANES)] = e[c]
                    s4 = (e[0] + e[1]) + (e[2] + e[3])
                    q4 = (e[0] * e[0] + e[1] * e[1]) + (e[2] * e[2] + e[3] * e[3])
                    sbuf[pl.ds(t * 17, LANES)] = s4
                    qbuf[pl.ds(t * 17, LANES)] = q4

            @plsc.parallel_loop(0, CT // LANES, unroll=2)
            def pass2(k):
                bvec = iota17 + k * (LANES * 17)
                stot = plsc.load_gather(sbuf, [bvec])
                qtot = plsc.load_gather(qbuf, [bvec])
                for j in range(1, LANES):
                    stot = stot + plsc.load_gather(sbuf, [bvec + j])
                    qtot = qtot + plsc.load_gather(qbuf, [bvec + j])
                mean = stot * (1.0 / DIM)
                var = qtot * (1.0 / DIM) - mean * mean
                rstd = _rsqrt(var + EPS)
                t0 = k * LANES
                for j in range(LANES):
                    t = t0 + j
                    mv = jnp.full((LANES,), mean[j], jnp.float32)
                    rv = jnp.full((LANES,), rstd[j], jnp.float32)
                    for c in range(NV):
                        e = rows_b[t, pl.ds(c * LANES, LANES)]
                        rows_b[t, pl.ds(c * LANES, LANES)] = (
                            (e - mv) * rv * g_regs[c] + b_regs[c])

        pltpu.sync_copy(x_hbm.at[base], idx0)
        fire_gather(idx0, rows0, gsem0)

        def pair_body(p, carry):
            cA = base + 2 * p
            cB = cA + 1
            drain_gather(idx0, rows0, gsem0)

            @pl.when(p > 0)
            def _():
                pltpu.make_async_copy(rows1, out_hbm.at[cB - 2], osem1).wait()

            pltpu.sync_copy(x_hbm.at[cB], idx1)
            fire_gather(idx1, rows1, gsem1)
            compute(rows0)
            pltpu.async_copy(rows0, out_hbm.at[cA], osem0)
            drain_gather(idx1, rows1, gsem1)

            @pl.when(p < NP - 1)
            def _():
                pltpu.make_async_copy(rows0, out_hbm.at[cA], osem0).wait()
                pltpu.sync_copy(x_hbm.at[cA + 2], idx0)
                fire_gather(idx0, rows0, gsem0)

            compute(rows1)
            pltpu.async_copy(rows1, out_hbm.at[cB], osem1)
            return carry

        lax.fori_loop(0, NP, pair_body, 0)
        pltpu.make_async_copy(rows0, out_hbm.at[base + CPW - 2], osem0).wait()
        pltpu.make_async_copy(rows1, out_hbm.at[base + CPW - 1], osem1).wait()

    out = sc_fn(x, W, P, gamma, beta)
    return out.reshape(B, S, DIM)


# speedup vs baseline: 1.9858x; 1.0490x over previous
import functools

import jax
import jax.numpy as jnp
from jax import lax
from jax.experimental import pallas as pl
from jax.experimental.pallas import tpu as pltpu
from jax.experimental.pallas import tpu_sc as plsc

DIM = 64
EPS = 1e-12
B, S = 4096, 200
NC, NS = 2, 16
NW = NC * NS
CHUNK_SEQ = 2
CT = CHUNK_SEQ * S
NCHUNK = B // CHUNK_SEQ
CPW = NCHUNK // NW
NP = CPW // 2
NIDX = 4
IDXW = CT // NIDX
LANES = 16
NV = DIM // LANES


def _rsqrt(v):
    vi = lax.bitcast_convert_type(v, jnp.int32)
    yi = jnp.int32(0x5F3759DF) - lax.shift_right_arithmetic(vi, 1)
    y = lax.bitcast_convert_type(yi, jnp.float32)
    h = v * 0.5
    for _ in range(3):
        y = y * (1.5 - h * y * y)
    return y


def kernel(x, W, P, gamma, beta):
    x = x.astype(jnp.int32).reshape(NCHUNK, NIDX, IDXW)
    mesh = plsc.VectorSubcoreMesh(core_axis_name="c", subcore_axis_name="s")

    @functools.partial(
        pl.kernel,
        out_type=jax.ShapeDtypeStruct((NCHUNK, CT, DIM), jnp.float32),
        mesh=mesh,
        scratch_types=[
            pltpu.VMEM((CPW, NIDX, IDXW), jnp.int32),
            pltpu.VMEM((CT, DIM), jnp.float32),
            pltpu.VMEM((CT, DIM), jnp.float32),
            pltpu.VMEM((S, DIM), jnp.float32),
            pltpu.VMEM((DIM,), jnp.float32),
            pltpu.VMEM((DIM,), jnp.float32),
            pltpu.VMEM((CT * 17,), jnp.float32),
            pltpu.VMEM((CT * 17,), jnp.float32),
            pltpu.SemaphoreType.DMA,
            pltpu.SemaphoreType.DMA,
            pltpu.SemaphoreType.DMA,
            pltpu.SemaphoreType.DMA,
        ],
        compiler_params=pltpu.CompilerParams(needs_layout_passes=False,
                                             use_tc_tiling_on_sc=False),
    )
    def sc_fn(x_hbm, w_hbm, p_hbm, g_hbm, b_hbm, out_hbm,
              idx_all, rows0, rows1, p_v, g_v, b_v,
              sbuf, qbuf,
              gsem0, gsem1, osem0, osem1):
        wid = lax.axis_index("s") * NC + lax.axis_index("c")
        base = wid * CPW
        pltpu.sync_copy(x_hbm.at[pl.ds(base, CPW)], idx_all)
        pltpu.sync_copy(p_hbm, p_v)
        pltpu.sync_copy(g_hbm, g_v)
        pltpu.sync_copy(b_hbm, b_v)
        g_regs = [g_v[pl.ds(c * LANES, LANES)] for c in range(NV)]
        b_regs = [b_v[pl.ds(c * LANES, LANES)] for c in range(NV)]

        def fire_gather(g, rows_b, sem):
            for j in range(NIDX):
                pltpu.async_copy(w_hbm.at[idx_all.at[g, j]],
                                 rows_b.at[pl.ds(j * IDXW, IDXW)], sem)

        def drain_gather(g, rows_b, sem):
            for j in range(NIDX):
                pltpu.make_async_copy(w_hbm.at[idx_all.at[g, j]],
                                      rows_b.at[pl.ds(j * IDXW, IDXW)],
                                      sem).wait()

        iota17 = lax.iota(jnp.int32, LANES) * 17

        def compute(rows_b):
            @plsc.parallel_loop(0, S, unroll=8)
            def pass1(si):
                p = [p_v[si, pl.ds(c * LANES, LANES)] for c in range(NV)]
                for r2 in range(CHUNK_SEQ):
                    t = r2 * S + si
                    e = [rows_b[t, pl.ds(c * LANES, LANES)] + p[c]
                         for c in range(NV)]
                    for c in range(NV):
                        rows_b[t, pl.ds(c * LANES, LANES)] = e[c]
                    s4 = (e[0] + e[1]) + (e[2] + e[3])
                    q4 = (e[0] * e[0] + e[1] * e[1]) + (e[2] * e[2] + e[3] * e[3])
                    sbuf[pl.ds(t * 17, LANES)] = s4
                    qbuf[pl.ds(t * 17, LANES)] = q4

            @plsc.parallel_loop(0, CT // LANES, unroll=2)
            def pass2(k):
                bvec = iota17 + k * (LANES * 17)
                stot = plsc.load_gather(sbuf, [bvec])
                qtot = plsc.load_gather(qbuf, [bvec])
                for j in range(1, LANES):
                    stot = stot + plsc.load_gather(sbuf, [bvec + j])
                    qtot = qtot + plsc.load_gather(qbuf, [bvec + j])
                mean = stot * (1.0 / DIM)
                var = qtot * (1.0 / DIM) - mean * mean
                rstd = _rsqrt(var + EPS)
                t0 = k * LANES
                for j in range(LANES):
                    t = t0 + j
                    mv = jnp.full((LANES,), mean[j], jnp.float32)
                    rv = jnp.full((LANES,), rstd[j], jnp.float32)
                    for c in range(NV):
                        e = rows_b[t, pl.ds(c * LANES, LANES)]
                        rows_b[t, pl.ds(c * LANES, LANES)] = (
                            (e - mv) * rv * g_regs[c] + b_regs[c])

        fire_gather(0, rows0, gsem0)

        def pair_body(p, carry):
            gA = 2 * p
            gB = gA + 1
            cA = base + gA
            cB = base + gB
            drain_gather(gA, rows0, gsem0)

            @pl.when(p > 0)
            def _():
                pltpu.make_async_copy(rows1, out_hbm.at[cB - 2], osem1).wait()

            fire_gather(gB, rows1, gsem1)
            compute(rows0)
            pltpu.async_copy(rows0, out_hbm.at[cA], osem0)
            drain_gather(gB, rows1, gsem1)

            @pl.when(p < NP - 1)
            def _():
                pltpu.make_async_copy(rows0, out_hbm.at[cA], osem0).wait()
                fire_gather(gA + 2, rows0, gsem0)

            compute(rows1)
            pltpu.async_copy(rows1, out_hbm.at[cB], osem1)
            return carry

        lax.fori_loop(0, NP, pair_body, 0)
        pltpu.make_async_copy(rows0, out_hbm.at[base + CPW - 2], osem0).wait()
        pltpu.make_async_copy(rows1, out_hbm.at[base + CPW - 1], osem1).wait()

    out = sc_fn(x, W, P, gamma, beta)
    return out.reshape(B, S, DIM)


# speedup vs baseline: 2.0673x; 1.0410x over previous
import functools

import jax
import jax.numpy as jnp
from jax import lax
from jax.experimental import pallas as pl
from jax.experimental.pallas import tpu as pltpu
from jax.experimental.pallas import tpu_sc as plsc

DIM = 64
EPS = 1e-12
B, S = 4096, 200
NC, NS = 2, 16
NW = NC * NS
CHUNK_SEQ = 2
CT = CHUNK_SEQ * S
NCHUNK = B // CHUNK_SEQ
CPW = NCHUNK // NW
NP = CPW // 2
NIDX = 4
IDXW = CT // NIDX
LANES = 16
NV = DIM // LANES
GPC = CT // LANES


def _rsqrt(v):
    vi = lax.bitcast_convert_type(v, jnp.int32)
    yi = jnp.int32(0x5F3759DF) - lax.shift_right_arithmetic(vi, 1)
    y = lax.bitcast_convert_type(yi, jnp.float32)
    h = v * 0.5
    for _ in range(3):
        y = y * (1.5 - h * y * y)
    return y


def kernel(x, W, P, gamma, beta):
    x = x.astype(jnp.int32).reshape(NCHUNK, NIDX, IDXW)
    mesh = plsc.VectorSubcoreMesh(core_axis_name="c", subcore_axis_name="s")

    @functools.partial(
        pl.kernel,
        out_type=jax.ShapeDtypeStruct((NCHUNK, S, 2 * DIM), jnp.float32),
        mesh=mesh,
        scratch_types=[
            pltpu.VMEM((NIDX, IDXW), jnp.int32),
            pltpu.VMEM((NIDX, IDXW), jnp.int32),
            pltpu.VMEM((CT, DIM), jnp.float32),
            pltpu.VMEM((S, 2 * DIM), jnp.float32),
            pltpu.VMEM((S, 2 * DIM), jnp.float32),
            pltpu.VMEM((S, DIM), jnp.float32),
            pltpu.VMEM((DIM,), jnp.float32),
            pltpu.VMEM((DIM,), jnp.float32),
            pltpu.VMEM((CT * 17,), jnp.float32),
            pltpu.VMEM((CT * 17,), jnp.float32),
            pltpu.SemaphoreType.DMA,
            pltpu.SemaphoreType.DMA,
            pltpu.SemaphoreType.DMA,
            pltpu.SemaphoreType.DMA,
            pltpu.SemaphoreType.DMA,
        ],
        compiler_params=pltpu.CompilerParams(needs_layout_passes=False,
                                             use_tc_tiling_on_sc=False),
    )
    def sc_fn(x_hbm, w_hbm, p_hbm, g_hbm, b_hbm, out_hbm,
              idx0, idx1, gbuf, wbuf0, wbuf1, p_v, g_v, b_v,
              sbuf, qbuf, gsem, osem0, osem1, isem0, isem1):
        wid = lax.axis_index("s") * NC + lax.axis_index("c")
        base = wid * CPW
        pltpu.sync_copy(p_hbm, p_v)
        pltpu.sync_copy(g_hbm, g_v)
        pltpu.sync_copy(b_hbm, b_v)
        g_regs = [g_v[pl.ds(c * LANES, LANES)] for c in range(NV)]
        b_regs = [b_v[pl.ds(c * LANES, LANES)] for c in range(NV)]

        def fire_gather(idx_b):
            for j in range(NIDX):
                pltpu.async_copy(w_hbm.at[idx_b.at[j]],
                                 gbuf.at[pl.ds(j * IDXW, IDXW)], gsem)

        def drain_gather(idx_b):
            for j in range(NIDX):
                pltpu.make_async_copy(w_hbm.at[idx_b.at[j]],
                                      gbuf.at[pl.ds(j * IDXW, IDXW)],
                                      gsem).wait()

        def fire_idx(cg, idx_b, isem):
            pltpu.async_copy(x_hbm.at[cg], idx_b, isem)

        def drain_idx(cg, idx_b, isem):
            pltpu.make_async_copy(x_hbm.at[cg], idx_b, isem).wait()

        iota17 = lax.iota(jnp.int32, LANES) * 17

        def pass1(wbuf):
            @plsc.parallel_loop(0, S, unroll=4)
            def body(si):
                p = [p_v[si, pl.ds(c * LANES, LANES)] for c in range(NV)]
                row = lax.shift_right_logical(si, 1)
                col = (si & 1) * DIM
                for r2 in range(CHUNK_SEQ):
                    t = r2 * S + si
                    e = [gbuf[t, pl.ds(c * LANES, LANES)] + p[c]
                         for c in range(NV)]
                    for c in range(NV):
                        wbuf[r2 * (S // 2) + row,
                             pl.ds(col + c * LANES, LANES)] = e[c]
                    s4 = (e[0] + e[1]) + (e[2] + e[3])
                    q4 = (e[0] * e[0] + e[1] * e[1]) + (e[2] * e[2] + e[3] * e[3])
                    sbuf[pl.ds(t * 17, LANES)] = s4
                    qbuf[pl.ds(t * 17, LANES)] = q4

        def pass2(wbuf):
            @plsc.parallel_loop(0, GPC, unroll=2)
            def body(k):
                bvec = iota17 + k * (LANES * 17)
                stot = plsc.load_gather(sbuf, [bvec])
                qtot = plsc.load_gather(qbuf, [bvec])
                for j in range(1, LANES):
                    stot = stot + plsc.load_gather(sbuf, [bvec + j])
                    qtot = qtot + plsc.load_gather(qbuf, [bvec + j])
                mean = stot * (1.0 / DIM)
                var = qtot * (1.0 / DIM) - mean * mean
                rstd = _rsqrt(var + EPS)
                row0 = k * (LANES // 2)
                for j in range(LANES):
                    mv = jnp.full((LANES,), mean[j], jnp.float32)
                    rv = jnp.full((LANES,), rstd[j], jnp.float32)
                    col = (j & 1) * DIM
                    for c in range(NV):
                        e = wbuf[row0 + j // 2, pl.ds(col + c * LANES, LANES)]
                        wbuf[row0 + j // 2, pl.ds(col + c * LANES, LANES)] = (
                            (e - mv) * rv * g_regs[c] + b_regs[c])

        pltpu.sync_copy(x_hbm.at[base], idx0)
        fire_gather(idx0)
        fire_idx(base + 1, idx1, isem1)

        def pair_body(p, carry):
            gA = 2 * p
            cA = base + gA
            cB = cA + 1
            drain_gather(idx0)

            @pl.when(p < NP - 1)
            def _():
                fire_idx(cA + 2, idx0, isem0)

            @pl.when(p > 0)
            def _():
                pltpu.make_async_copy(wbuf0, out_hbm.at[cA - 2], osem0).wait()

            pass1(wbuf0)
            drain_idx(cB, idx1, isem1)
            fire_gather(idx1)
            pass2(wbuf0)
            pltpu.async_copy(wbuf0, out_hbm.at[cA], osem0)

            drain_gather(idx1)

            @pl.when(p < NP - 1)
            def _():
                fire_idx(cB + 2, idx1, isem1)

            @pl.when(p > 0)
            def _():
                pltpu.make_async_copy(wbuf1, out_hbm.at[cB - 2], osem1).wait()

            pass1(wbuf1)

            @pl.when(p < NP - 1)
            def _():
                drain_idx(cA + 2, idx0, isem0)
                fire_gather(idx0)

            pass2(wbuf1)
            pltpu.async_copy(wbuf1, out_hbm.at[cB], osem1)
            return carry

        lax.fori_loop(0, NP, pair_body, 0)
        pltpu.make_async_copy(wbuf0, out_hbm.at[base + CPW - 2], osem0).wait()
        pltpu.make_async_copy(wbuf1, out_hbm.at[base + CPW - 1], osem1).wait()

    out = sc_fn(x, W, P, gamma, beta)
    return out.reshape(B, S, DIM)
